# Initial kernel scaffold; baseline (speedup 1.0000x reference)
import jax
import jax.numpy as jnp
from jax.experimental import pallas as pl


def kernel(g, user_embeddings, item_v_feat, item_t_feat, t_W, t_b, t_bn_g, t_bn_b, t_a, v_W, v_b, v_bn_g, v_bn_b, v_a):
    raise NotImplementedError("write your pallas kernel here")


# speedup vs baseline: 9.5921x; 9.5921x over previous
import functools
import math

import jax
import jax.numpy as jnp
from jax import lax
from jax.experimental import pallas as pl
from jax.experimental.pallas import tpu as pltpu
from jax.experimental.pallas import tpu_sc as plsc

N_USERS = 25000
N_ITEMS = 25000
N_NODES = 50000
N_EDGES = 800000
DIM = 64
HALF = 32
KSTEPS = 4
ALPHA = 0.1
BETA = 0.9
BN_EPS = 1e-5
DENOM = BETA ** KSTEPS + ALPHA * sum(BETA ** i for i in range(KSTEPS))

NSC = 2
NT = 16
NPAD = 51200
TNODES = 3200
NCH = 128
ZR = 32
EPAD = 819200
ECH = 400
GARBAGE = 50000


def _sc_body(src_ref, dst_ref, x2_ref, ax2_ref_in, out_ref, gtab_ref,
             af, cbf, bdf, sidx, didx, grow, xb, accb, gb, zb, acc, sem):
    c = lax.axis_index("c")
    t = lax.axis_index("s")
    coff = c * NPAD
    ebase = t * (ECH * 128)
    gbase = coff + t * TNODES

    zeros16f = jnp.zeros((16,), jnp.float32)
    ones16 = jnp.full((16,), 1.0, jnp.float32)
    iota16 = lax.iota(jnp.int32, 16)
    zeros16i = jnp.zeros((16,), jnp.int32)

    @pl.loop(0, ZR)
    def _zero_zb(r):
        zb[r, pl.ds(0, 16)] = zeros16f
        zb[r, pl.ds(16, 16)] = zeros16f

    @pl.loop(0, 128)
    def _ones_grow(r):
        grow[r, pl.ds(0, 16)] = ones16
        grow[r, pl.ds(16, 16)] = ones16

    def _zero_acc_chunk(ch):
        for q in range(NCH // ZR):
            pltpu.sync_copy(
                zb, acc.at[pl.ds(t * TNODES + ch * NCH + q * ZR, ZR)])

    def _zero_acc():
        @pl.loop(0, TNODES // NCH)
        def _z(ch):
            _zero_acc_chunk(ch)

    def _deg_pass(edge_ref, dest):
        _zero_acc()
        plsc.subcore_barrier()

        @pl.loop(0, ECH)
        def _scat(i):
            pltpu.sync_copy(edge_ref.at[pl.ds(ebase + i * 128, 128)], didx)
            pltpu.sync_copy(grow, acc.at[didx], add=True)

        plsc.subcore_barrier()

        @pl.loop(0, TNODES // NCH)
        def _extract(ch):
            pltpu.sync_copy(acc.at[pl.ds(t * TNODES + ch * NCH, NCH)], accb)
            for k in range(NCH // 16):
                v = plsc.load_gather(accb, [iota16 + k * 16, zeros16i])
                dest[pl.ds(ch * NCH + k * 16, 16)] = v

    _deg_pass(src_ref, af)
    _deg_pass(dst_ref, bdf)

    def _rsqrt(d):
        i = plsc.bitcast(d, jnp.int32)
        y = plsc.bitcast(0x5F3759DF - (i >> 1), jnp.float32)
        for _ in range(3):
            y = y * (1.5 - 0.5 * d * y * y)
        return y

    @pl.loop(0, TNODES // 16)
    def _scales(i):
        sl = pl.ds(i * 16, 16)
        av = _rsqrt(jnp.maximum(af[sl], 1.0))
        bv = _rsqrt(jnp.maximum(bdf[sl], 1.0))
        af[sl] = av
        cbf[sl] = BETA * av * bv
        bdf[sl] = (BETA / DENOM) * bv

    @pl.loop(0, TNODES // NCH)
    def _g0(ch):
        base = gbase + ch * NCH
        pltpu.sync_copy(x2_ref.at[pl.ds(base, NCH)], xb)

        @pl.loop(0, NCH)
        def _rows(j):
            ln = ch * NCH + j
            av = af[pl.ds(ln, 16)][0]
            for f in range(2):
                sl = pl.ds(16 * f, 16)
                gb[j, sl] = av * xb[j, sl]

        pltpu.sync_copy(gb, gtab_ref.at[pl.ds(base, NCH)])
        pltpu.sync_copy(gb, ax2_ref_in.at[pl.ds(base, NCH)])
        _zero_acc_chunk(ch)

    plsc.subcore_barrier()

    @pl.loop(0, KSTEPS)
    def _step(it):
        @pl.loop(0, ECH)
        def _edges(i):
            off = ebase + i * 128
            pltpu.sync_copy(src_ref.at[pl.ds(off, 128)], sidx)
            for j in range(8):
                sl = pl.ds(16 * j, 16)
                sidx[sl] = sidx[sl] + coff
            pltpu.sync_copy(dst_ref.at[pl.ds(off, 128)], didx)
            pltpu.async_copy(gtab_ref.at[sidx], grow, sem).wait()
            pltpu.sync_copy(grow, acc.at[didx], add=True)

        plsc.subcore_barrier()

        last = it == KSTEPS - 1

        @pl.loop(0, TNODES // NCH)
        def _nodes(ch):
            base = gbase + ch * NCH

            @pl.when(jnp.logical_not(last))
            def _():
                pltpu.sync_copy(ax2_ref_in.at[pl.ds(base, NCH)], xb)

            @pl.when(last)
            def _():
                pltpu.sync_copy(x2_ref.at[pl.ds(base, NCH)], xb)

            pltpu.sync_copy(acc.at[pl.ds(t * TNODES + ch * NCH, NCH)], accb)

            @pl.loop(0, NCH)
            def _rows(j):
                ln = ch * NCH + j
                cbv = cbf[pl.ds(ln, 16)][0]
                bdv = bdf[pl.ds(ln, 16)][0]
                for f in range(2):
                    sl = pl.ds(16 * f, 16)
                    un = ALPHA * xb[j, sl] + cbv * accb[j, sl]
                    ov = (ALPHA / DENOM) * xb[j, sl] + bdv * accb[j, sl]
                    gb[j, sl] = jnp.where(last, ov, un)

            @pl.when(jnp.logical_not(last))
            def _():
                pltpu.sync_copy(gb, gtab_ref.at[pl.ds(base, NCH)])

            @pl.when(last)
            def _():
                pltpu.sync_copy(gb, out_ref.at[pl.ds(base, NCH)])

            _zero_acc_chunk(ch)

        plsc.subcore_barrier()


def _sc_diffuse(src_p, dst_p, x2, ax_init):
    f32 = jnp.float32
    mesh = plsc.VectorSubcoreMesh(core_axis_name="c", subcore_axis_name="s")
    fn = pl.kernel(
        _sc_body,
        out_type=[
            jax.ShapeDtypeStruct((NSC * NPAD, HALF), f32),
            jax.ShapeDtypeStruct((NSC * NPAD, HALF), f32),
            jax.ShapeDtypeStruct((NSC * NPAD, HALF), f32),
        ],
        mesh=mesh,
        compiler_params=pltpu.CompilerParams(needs_layout_passes=False,
                                             use_tc_tiling_on_sc=False),
        scratch_types=[
            pltpu.VMEM((TNODES + 16,), f32),
            pltpu.VMEM((TNODES + 16,), f32),
            pltpu.VMEM((TNODES + 16,), f32),
            pltpu.VMEM((128,), jnp.int32),
            pltpu.VMEM((128,), jnp.int32),
            pltpu.VMEM((128, HALF), f32),
            pltpu.VMEM((NCH, HALF), f32),
            pltpu.VMEM((NCH, HALF), f32),
            pltpu.VMEM((NCH, HALF), f32),
            pltpu.VMEM((ZR, HALF), f32),
            pltpu.VMEM_SHARED((NPAD, HALF), f32),
            pltpu.SemaphoreType.DMA,
        ],
    )
    ax, out, _ = fn(src_p, dst_p, x2)
    del ax
    return out


def _enc_kernel(t_ref, v_ref, tW_ref, tb_ref, tg_ref, tbb_ref, ta_ref,
                vW_ref, vb_ref, vg_ref, vbb_ref, va_ref, out_ref):
    inv = 1.0 / math.sqrt(1.0 + BN_EPS)
    ht = jnp.dot(t_ref[...], tW_ref[...], preferred_element_type=jnp.float32)
    ht = (ht + tb_ref[...]) * (inv * tg_ref[...]) + tbb_ref[...]
    ht = jnp.where(ht > 0, ht, ta_ref[...] * ht)
    hv = jnp.dot(v_ref[...], vW_ref[...], preferred_element_type=jnp.float32)
    hv = (hv + vb_ref[...]) * (inv * vg_ref[...]) + vbb_ref[...]
    hv = jnp.where(hv > 0, hv, va_ref[...] * hv)
    out_ref[...] = ht + hv


def _encode(item_t_feat, item_v_feat, t_W, t_b, t_bn_g, t_bn_b, t_a,
            v_W, v_b, v_bn_g, v_bn_b, v_a):
    R = 1000
    grid = (N_ITEMS // R,)
    row = lambda i: (i, 0)
    fix = lambda i: (0, 0)
    return pl.pallas_call(
        _enc_kernel,
        grid=grid,
        in_specs=[
            pl.BlockSpec((R, 384), row),
            pl.BlockSpec((R, 512), row),
            pl.BlockSpec((384, DIM), fix),
            pl.BlockSpec((1, DIM), fix),
            pl.BlockSpec((1, DIM), fix),
            pl.BlockSpec((1, DIM), fix),
            pl.BlockSpec((1, 1), fix),
            pl.BlockSpec((512, DIM), fix),
            pl.BlockSpec((1, DIM), fix),
            pl.BlockSpec((1, DIM), fix),
            pl.BlockSpec((1, DIM), fix),
            pl.BlockSpec((1, 1), fix),
        ],
        out_specs=pl.BlockSpec((R, DIM), row),
        out_shape=jax.ShapeDtypeStruct((N_ITEMS, DIM), jnp.float32),
    )(item_t_feat, item_v_feat,
      t_W, t_b.reshape(1, DIM), t_bn_g.reshape(1, DIM),
      t_bn_b.reshape(1, DIM), t_a.reshape(1, 1),
      v_W, v_b.reshape(1, DIM), v_bn_g.reshape(1, DIM),
      v_bn_b.reshape(1, DIM), v_a.reshape(1, 1))


def kernel(g, user_embeddings, item_v_feat, item_t_feat, t_W, t_b, t_bn_g,
           t_bn_b, t_a, v_W, v_b, v_bn_g, v_bn_b, v_a):
    enc = _encode(item_t_feat, item_v_feat, t_W, t_b, t_bn_g, t_bn_b, t_a,
                  v_W, v_b, v_bn_g, v_bn_b, v_a)
    x = jnp.concatenate([user_embeddings, enc], axis=0)

    x2 = jnp.zeros((NSC * NPAD, HALF), jnp.float32)
    x2 = x2.at[:N_NODES].set(x[:, :HALF])
    x2 = x2.at[NPAD:NPAD + N_NODES].set(x[:, HALF:])

    pad = jnp.full((EPAD - N_EDGES,), GARBAGE, jnp.int32)
    src_p = jnp.concatenate([g[0], pad])
    dst_p = jnp.concatenate([g[1], pad])

    out = _sc_diffuse(src_p, dst_p, x2, None)
    return jnp.concatenate([out[:N_NODES], out[NPAD:NPAD + N_NODES]], axis=1)


# speedup vs baseline: 13.9488x; 1.4542x over previous
import functools
import math

import jax
import jax.numpy as jnp
from jax import lax
from jax.experimental import pallas as pl
from jax.experimental.pallas import tpu as pltpu
from jax.experimental.pallas import tpu_sc as plsc

N_USERS = 25000
N_ITEMS = 25000
N_NODES = 50000
N_EDGES = 800000
DIM = 64
HALF = 32
KSTEPS = 4
ALPHA = 0.1
BETA = 0.9
BN_EPS = 1e-5
DENOM = BETA ** KSTEPS + ALPHA * sum(BETA ** i for i in range(KSTEPS))

NSC = 2
NT = 16
NPAD = 51200
TNODES = 3200
NCH = 128
ZR = 32
EPAD = 819200
ECH = 400
GARBAGE = 50000


def _sc_body(src_ref, dst_ref, x2_ref, ax2_ref_in, out_ref, gtab_ref,
             af, cbf, bdf, sidx, didx, sidx1, didx1, grow, xb, accb, gb,
             zb, acc, sem, sem1):
    c = lax.axis_index("c")
    t = lax.axis_index("s")
    coff = c * NPAD
    ebase = t * (ECH * 128)
    gbase = coff + t * TNODES

    zeros16f = jnp.zeros((16,), jnp.float32)
    ones16 = jnp.full((16,), 1.0, jnp.float32)
    iota16 = lax.iota(jnp.int32, 16)
    zeros16i = jnp.zeros((16,), jnp.int32)

    @pl.loop(0, ZR)
    def _zero_zb(r):
        zb[r, pl.ds(0, 16)] = zeros16f
        zb[r, pl.ds(16, 16)] = zeros16f

    @pl.loop(0, 128)
    def _ones_grow(r):
        grow[r, pl.ds(0, 16)] = ones16
        grow[r, pl.ds(16, 16)] = ones16

    def _zero_acc_chunk(ch):
        for q in range(NCH // ZR):
            pltpu.sync_copy(
                zb, acc.at[pl.ds(t * TNODES + ch * NCH + q * ZR, ZR)])

    def _zero_acc():
        @pl.loop(0, TNODES // NCH)
        def _z(ch):
            _zero_acc_chunk(ch)

    def _deg_pass(edge_ref, dest):
        _zero_acc()
        plsc.subcore_barrier()

        @pl.loop(0, ECH)
        def _scat(i):
            pltpu.sync_copy(edge_ref.at[pl.ds(ebase + i * 128, 128)], didx)
            pltpu.sync_copy(grow, acc.at[didx], add=True)

        plsc.subcore_barrier()

        @pl.loop(0, TNODES // NCH)
        def _extract(ch):
            pltpu.sync_copy(acc.at[pl.ds(t * TNODES + ch * NCH, NCH)], accb)
            for k in range(NCH // 16):
                v = plsc.load_gather(accb, [iota16 + k * 16, zeros16i])
                dest[pl.ds(ch * NCH + k * 16, 16)] = v

    _deg_pass(src_ref, af)
    _deg_pass(dst_ref, bdf)

    def _rsqrt(d):
        i = plsc.bitcast(d, jnp.int32)
        y = plsc.bitcast(0x5F3759DF - (i >> 1), jnp.float32)
        for _ in range(3):
            y = y * (1.5 - 0.5 * d * y * y)
        return y

    @pl.loop(0, TNODES // 16)
    def _scales(i):
        sl = pl.ds(i * 16, 16)
        av = _rsqrt(jnp.maximum(af[sl], 1.0))
        bv = _rsqrt(jnp.maximum(bdf[sl], 1.0))
        af[sl] = av
        cbf[sl] = BETA * av * bv
        bdf[sl] = (BETA / DENOM) * bv

    @pl.loop(0, TNODES // NCH)
    def _g0(ch):
        base = gbase + ch * NCH
        pltpu.sync_copy(x2_ref.at[pl.ds(base, NCH)], xb)

        @pl.loop(0, NCH)
        def _rows(j):
            ln = ch * NCH + j
            av = af[pl.ds(ln, 16)][0]
            for f in range(2):
                sl = pl.ds(16 * f, 16)
                gb[j, sl] = av * xb[j, sl]

        pltpu.sync_copy(gb, gtab_ref.at[pl.ds(base, NCH)])
        pltpu.sync_copy(gb, ax2_ref_in.at[pl.ds(base, NCH)])
        _zero_acc_chunk(ch)

    plsc.subcore_barrier()

    def _edge_prep(i, sid, did):
        off = ebase + i * 128
        pltpu.sync_copy(src_ref.at[pl.ds(off, 128)], sid)
        for j in range(8):
            sl = pl.ds(16 * j, 16)
            sid[sl] = sid[sl] + coff
        pltpu.sync_copy(dst_ref.at[pl.ds(off, 128)], did)

    @pl.loop(0, KSTEPS)
    def _step(it):
        _edge_prep(0, sidx, didx)
        pltpu.async_copy(gtab_ref.at[sidx], grow, sem)

        @pl.loop(0, ECH // 2)
        def _edges(i2):
            i = i2 * 2
            _edge_prep(i + 1, sidx1, didx1)
            pltpu.async_copy(gtab_ref.at[sidx1], xb, sem1)
            pltpu.make_async_copy(gtab_ref.at[sidx], grow, sem).wait()
            pltpu.sync_copy(grow, acc.at[didx], add=True)

            @pl.when(i2 < ECH // 2 - 1)
            def _():
                _edge_prep(i + 2, sidx, didx)
                pltpu.async_copy(gtab_ref.at[sidx], grow, sem)

            pltpu.make_async_copy(gtab_ref.at[sidx1], xb, sem1).wait()
            pltpu.sync_copy(xb, acc.at[didx1], add=True)

        plsc.subcore_barrier()

        last = it == KSTEPS - 1

        @pl.loop(0, TNODES // NCH)
        def _nodes(ch):
            base = gbase + ch * NCH

            @pl.when(jnp.logical_not(last))
            def _():
                pltpu.sync_copy(ax2_ref_in.at[pl.ds(base, NCH)], xb)

            @pl.when(last)
            def _():
                pltpu.sync_copy(x2_ref.at[pl.ds(base, NCH)], xb)

            pltpu.sync_copy(acc.at[pl.ds(t * TNODES + ch * NCH, NCH)], accb)

            @pl.loop(0, NCH)
            def _rows(j):
                ln = ch * NCH + j
                cbv = cbf[pl.ds(ln, 16)][0]
                bdv = bdf[pl.ds(ln, 16)][0]
                for f in range(2):
                    sl = pl.ds(16 * f, 16)
                    un = ALPHA * xb[j, sl] + cbv * accb[j, sl]
                    ov = (ALPHA / DENOM) * xb[j, sl] + bdv * accb[j, sl]
                    gb[j, sl] = jnp.where(last, ov, un)

            @pl.when(jnp.logical_not(last))
            def _():
                pltpu.sync_copy(gb, gtab_ref.at[pl.ds(base, NCH)])

            @pl.when(last)
            def _():
                pltpu.sync_copy(gb, out_ref.at[pl.ds(base, NCH)])

            _zero_acc_chunk(ch)

        plsc.subcore_barrier()


def _sc_diffuse(src_p, dst_p, x2, ax_init):
    f32 = jnp.float32
    mesh = plsc.VectorSubcoreMesh(core_axis_name="c", subcore_axis_name="s")
    fn = pl.kernel(
        _sc_body,
        out_type=[
            jax.ShapeDtypeStruct((NSC * NPAD, HALF), f32),
            jax.ShapeDtypeStruct((NSC * NPAD, HALF), f32),
            jax.ShapeDtypeStruct((NSC * NPAD, HALF), f32),
        ],
        mesh=mesh,
        compiler_params=pltpu.CompilerParams(needs_layout_passes=False,
                                             use_tc_tiling_on_sc=False),
        scratch_types=[
            pltpu.VMEM((TNODES + 16,), f32),
            pltpu.VMEM((TNODES + 16,), f32),
            pltpu.VMEM((TNODES + 16,), f32),
            pltpu.VMEM((128,), jnp.int32),
            pltpu.VMEM((128,), jnp.int32),
            pltpu.VMEM((128,), jnp.int32),
            pltpu.VMEM((128,), jnp.int32),
            pltpu.VMEM((128, HALF), f32),
            pltpu.VMEM((NCH, HALF), f32),
            pltpu.VMEM((NCH, HALF), f32),
            pltpu.VMEM((NCH, HALF), f32),
            pltpu.VMEM((ZR, HALF), f32),
            pltpu.VMEM_SHARED((NPAD, HALF), f32),
            pltpu.SemaphoreType.DMA,
            pltpu.SemaphoreType.DMA,
        ],
    )
    ax, out, _ = fn(src_p, dst_p, x2)
    del ax
    return out


def _enc_kernel(t_ref, v_ref, tW_ref, tb_ref, tg_ref, tbb_ref, ta_ref,
                vW_ref, vb_ref, vg_ref, vbb_ref, va_ref, out_ref):
    inv = 1.0 / math.sqrt(1.0 + BN_EPS)
    ht = jnp.dot(t_ref[...], tW_ref[...], preferred_element_type=jnp.float32)
    ht = (ht + tb_ref[...]) * (inv * tg_ref[...]) + tbb_ref[...]
    ht = jnp.where(ht > 0, ht, ta_ref[...] * ht)
    hv = jnp.dot(v_ref[...], vW_ref[...], preferred_element_type=jnp.float32)
    hv = (hv + vb_ref[...]) * (inv * vg_ref[...]) + vbb_ref[...]
    hv = jnp.where(hv > 0, hv, va_ref[...] * hv)
    out_ref[...] = ht + hv


def _encode(item_t_feat, item_v_feat, t_W, t_b, t_bn_g, t_bn_b, t_a,
            v_W, v_b, v_bn_g, v_bn_b, v_a):
    R = 1000
    grid = (N_ITEMS // R,)
    row = lambda i: (i, 0)
    fix = lambda i: (0, 0)
    return pl.pallas_call(
        _enc_kernel,
        grid=grid,
        in_specs=[
            pl.BlockSpec((R, 384), row),
            pl.BlockSpec((R, 512), row),
            pl.BlockSpec((384, DIM), fix),
            pl.BlockSpec((1, DIM), fix),
            pl.BlockSpec((1, DIM), fix),
            pl.BlockSpec((1, DIM), fix),
            pl.BlockSpec((1, 1), fix),
            pl.BlockSpec((512, DIM), fix),
            pl.BlockSpec((1, DIM), fix),
            pl.BlockSpec((1, DIM), fix),
            pl.BlockSpec((1, DIM), fix),
            pl.BlockSpec((1, 1), fix),
        ],
        out_specs=pl.BlockSpec((R, DIM), row),
        out_shape=jax.ShapeDtypeStruct((N_ITEMS, DIM), jnp.float32),
    )(item_t_feat, item_v_feat,
      t_W, t_b.reshape(1, DIM), t_bn_g.reshape(1, DIM),
      t_bn_b.reshape(1, DIM), t_a.reshape(1, 1),
      v_W, v_b.reshape(1, DIM), v_bn_g.reshape(1, DIM),
      v_bn_b.reshape(1, DIM), v_a.reshape(1, 1))


def kernel(g, user_embeddings, item_v_feat, item_t_feat, t_W, t_b, t_bn_g,
           t_bn_b, t_a, v_W, v_b, v_bn_g, v_bn_b, v_a):
    enc = _encode(item_t_feat, item_v_feat, t_W, t_b, t_bn_g, t_bn_b, t_a,
                  v_W, v_b, v_bn_g, v_bn_b, v_a)
    x = jnp.concatenate([user_embeddings, enc], axis=0)

    x2 = jnp.zeros((NSC * NPAD, HALF), jnp.float32)
    x2 = x2.at[:N_NODES].set(x[:, :HALF])
    x2 = x2.at[NPAD:NPAD + N_NODES].set(x[:, HALF:])

    pad = jnp.full((EPAD - N_EDGES,), GARBAGE, jnp.int32)
    src_p = jnp.concatenate([g[0], pad])
    dst_p = jnp.concatenate([g[1], pad])

    out = _sc_diffuse(src_p, dst_p, x2, None)
    return jnp.concatenate([out[:N_NODES], out[NPAD:NPAD + N_NODES]], axis=1)


# speedup vs baseline: 18.3778x; 1.3175x over previous
import functools
import math

import jax
import jax.numpy as jnp
from jax import lax
from jax.experimental import pallas as pl
from jax.experimental.pallas import tpu as pltpu
from jax.experimental.pallas import tpu_sc as plsc

N_USERS = 25000
N_ITEMS = 25000
N_NODES = 50000
N_EDGES = 800000
DIM = 64
HALF = 32
KSTEPS = 4
ALPHA = 0.1
BETA = 0.9
BN_EPS = 1e-5
DENOM = BETA ** KSTEPS + ALPHA * sum(BETA ** i for i in range(KSTEPS))

NSC = 2
NT = 16
NPAD = 51200
TNODES = 3200
NCH = 128
ZR = 32
EPAD = 819200
ECH = 400
GARBAGE = 50000


def _sc_body(src_ref, dst_ref, x2_ref, ax2_ref_in, out_ref, gtab_ref,
             af, cbf, bdf, sidx, didx, sidx1, didx1, sidx2, didx2, sidx3,
             didx3, grow, xb, accb, gb, zb, acc, sem, sem1, isem, ssem0,
             ssem1):
    c = lax.axis_index("c")
    t = lax.axis_index("s")
    coff = c * NPAD
    ebase = t * (ECH * 128)
    gbase = coff + t * TNODES

    zeros16f = jnp.zeros((16,), jnp.float32)
    ones16 = jnp.full((16,), 1.0, jnp.float32)
    iota16 = lax.iota(jnp.int32, 16)
    zeros16i = jnp.zeros((16,), jnp.int32)

    @pl.loop(0, ZR)
    def _zero_zb(r):
        zb[r, pl.ds(0, 16)] = zeros16f
        zb[r, pl.ds(16, 16)] = zeros16f

    @pl.loop(0, 128)
    def _ones_grow(r):
        grow[r, pl.ds(0, 16)] = ones16
        grow[r, pl.ds(16, 16)] = ones16

    def _zero_acc_chunk(ch):
        for q in range(NCH // ZR):
            pltpu.sync_copy(
                zb, acc.at[pl.ds(t * TNODES + ch * NCH + q * ZR, ZR)])

    def _zero_acc():
        @pl.loop(0, TNODES // NCH)
        def _z(ch):
            _zero_acc_chunk(ch)

    SETS = ((sidx, didx), (sidx1, didx1), (sidx2, didx2), (sidx3, didx3))
    GROW = (grow, gb)
    GSEM = (sem, sem1)
    SSEM = (ssem0, ssem1)

    def _idx_start(edge_ref, i, s, both):
        off = ebase + i * 128
        if both:
            pltpu.async_copy(src_ref.at[pl.ds(off, 128)], SETS[s][0], isem)
            pltpu.async_copy(dst_ref.at[pl.ds(off, 128)], SETS[s][1], isem)
        else:
            pltpu.async_copy(edge_ref.at[pl.ds(off, 128)], SETS[s][1], isem)

    def _idx_wait(edge_ref, i, s, both):
        off = ebase + i * 128
        if both:
            pltpu.make_async_copy(
                src_ref.at[pl.ds(off, 128)], SETS[s][0], isem).wait()
            pltpu.make_async_copy(
                dst_ref.at[pl.ds(off, 128)], SETS[s][1], isem).wait()
        else:
            pltpu.make_async_copy(
                edge_ref.at[pl.ds(off, 128)], SETS[s][1], isem).wait()

    def _scat_wait(g, did, ss):
        pltpu.make_async_copy(g, acc.at[did], ss).wait()

    def _deg_pass(edge_ref, dest):
        _zero_acc()
        plsc.subcore_barrier()

        _idx_start(edge_ref, 0, 0, False)
        _idx_start(edge_ref, 1, 1, False)

        @pl.loop(0, ECH // 4)
        def _scat(i4):
            for k in range(4):
                i = i4 * 4 + k
                did = SETS[k][1]
                ss = SSEM[k % 2]
                _idx_wait(edge_ref, i, k, False)
                if k < 2:
                    @pl.when(i4 > 0)
                    def _():
                        _scat_wait(grow, did, ss)
                else:
                    _scat_wait(grow, did, ss)
                pltpu.async_copy(grow, acc.at[did], ss, add=True)
                if k < 2:
                    _idx_start(edge_ref, i + 2, (k + 2) % 4, False)
                else:
                    @pl.when(i4 < ECH // 4 - 1)
                    def _():
                        _idx_start(edge_ref, i + 2, (k + 2) % 4, False)

        _scat_wait(grow, SETS[2][1], SSEM[0])
        _scat_wait(grow, SETS[3][1], SSEM[1])
        plsc.subcore_barrier()

        @pl.loop(0, TNODES // NCH)
        def _extract(ch):
            pltpu.sync_copy(acc.at[pl.ds(t * TNODES + ch * NCH, NCH)], accb)
            for k in range(NCH // 16):
                v = plsc.load_gather(accb, [iota16 + k * 16, zeros16i])
                dest[pl.ds(ch * NCH + k * 16, 16)] = v

    _deg_pass(src_ref, af)
    _deg_pass(dst_ref, bdf)

    def _rsqrt(d):
        i = plsc.bitcast(d, jnp.int32)
        y = plsc.bitcast(0x5F3759DF - (i >> 1), jnp.float32)
        for _ in range(3):
            y = y * (1.5 - 0.5 * d * y * y)
        return y

    @pl.loop(0, TNODES // 16)
    def _scales(i):
        sl = pl.ds(i * 16, 16)
        av = _rsqrt(jnp.maximum(af[sl], 1.0))
        bv = _rsqrt(jnp.maximum(bdf[sl], 1.0))
        af[sl] = av
        cbf[sl] = BETA * av * bv
        bdf[sl] = (BETA / DENOM) * bv

    @pl.loop(0, TNODES // NCH)
    def _g0(ch):
        base = gbase + ch * NCH
        pltpu.sync_copy(x2_ref.at[pl.ds(base, NCH)], xb)

        @pl.loop(0, NCH)
        def _rows(j):
            ln = ch * NCH + j
            av = af[pl.ds(ln, 16)][0]
            for f in range(2):
                sl = pl.ds(16 * f, 16)
                gb[j, sl] = av * xb[j, sl]

        pltpu.sync_copy(gb, gtab_ref.at[pl.ds(base, NCH)])
        pltpu.sync_copy(gb, ax2_ref_in.at[pl.ds(base, NCH)])
        _zero_acc_chunk(ch)

    plsc.subcore_barrier()

    @pl.loop(0, KSTEPS)
    def _step(it):
        _idx_start(src_ref, 0, 0, True)
        _idx_start(src_ref, 1, 1, True)

        @pl.loop(0, ECH // 4)
        def _edges(i4):
            for k in range(4):
                i = i4 * 4 + k
                sid, did = SETS[k]
                g = GROW[k % 2]
                _idx_wait(src_ref, i, k, True)
                for j in range(8):
                    sl = pl.ds(16 * j, 16)
                    sid[sl] = sid[sl] + coff
                if k < 2:
                    @pl.when(i4 > 0)
                    def _():
                        _scat_wait(g, did, SSEM[k % 2])
                else:
                    _scat_wait(g, did, SSEM[k % 2])
                pltpu.async_copy(gtab_ref.at[sid], g, GSEM[k % 2])
                if k < 2:
                    _idx_start(src_ref, i + 2, (k + 2) % 4, True)
                else:
                    @pl.when(i4 < ECH // 4 - 1)
                    def _():
                        _idx_start(src_ref, i + 2, (k + 2) % 4, True)
                pg = GROW[(k + 1) % 2]
                psid, pdid = SETS[(k + 3) % 4]

                def _finish_prev(pg=pg, psid=psid, pdid=pdid, k=k):
                    pltpu.make_async_copy(gtab_ref.at[psid], pg,
                                          GSEM[(k + 1) % 2]).wait()
                    pltpu.async_copy(pg, acc.at[pdid],
                                     SSEM[(k + 1) % 2], add=True)

                if k == 0:
                    pl.when(i4 > 0)(_finish_prev)
                else:
                    _finish_prev()

        pltpu.make_async_copy(gtab_ref.at[SETS[3][0]], GROW[1],
                              GSEM[1]).wait()
        pltpu.async_copy(GROW[1], acc.at[SETS[3][1]], SSEM[1], add=True)
        _scat_wait(GROW[0], SETS[2][1], SSEM[0])
        _scat_wait(GROW[1], SETS[3][1], SSEM[1])
        plsc.subcore_barrier()

        last = it == KSTEPS - 1

        @pl.loop(0, TNODES // NCH)
        def _nodes(ch):
            base = gbase + ch * NCH

            @pl.when(jnp.logical_not(last))
            def _():
                pltpu.sync_copy(ax2_ref_in.at[pl.ds(base, NCH)], xb)

            @pl.when(last)
            def _():
                pltpu.sync_copy(x2_ref.at[pl.ds(base, NCH)], xb)

            pltpu.sync_copy(acc.at[pl.ds(t * TNODES + ch * NCH, NCH)], accb)

            @pl.loop(0, NCH)
            def _rows(j):
                ln = ch * NCH + j
                cbv = cbf[pl.ds(ln, 16)][0]
                bdv = bdf[pl.ds(ln, 16)][0]
                for f in range(2):
                    sl = pl.ds(16 * f, 16)
                    un = ALPHA * xb[j, sl] + cbv * accb[j, sl]
                    ov = (ALPHA / DENOM) * xb[j, sl] + bdv * accb[j, sl]
                    gb[j, sl] = jnp.where(last, ov, un)

            @pl.when(jnp.logical_not(last))
            def _():
                pltpu.sync_copy(gb, gtab_ref.at[pl.ds(base, NCH)])

            @pl.when(last)
            def _():
                pltpu.sync_copy(gb, out_ref.at[pl.ds(base, NCH)])

            _zero_acc_chunk(ch)

        plsc.subcore_barrier()


def _sc_diffuse(src_p, dst_p, x2, ax_init):
    f32 = jnp.float32
    mesh = plsc.VectorSubcoreMesh(core_axis_name="c", subcore_axis_name="s")
    fn = pl.kernel(
        _sc_body,
        out_type=[
            jax.ShapeDtypeStruct((NSC * NPAD, HALF), f32),
            jax.ShapeDtypeStruct((NSC * NPAD, HALF), f32),
            jax.ShapeDtypeStruct((NSC * NPAD, HALF), f32),
        ],
        mesh=mesh,
        compiler_params=pltpu.CompilerParams(needs_layout_passes=False,
                                             use_tc_tiling_on_sc=False),
        scratch_types=[
            pltpu.VMEM((TNODES + 16,), f32),
            pltpu.VMEM((TNODES + 16,), f32),
            pltpu.VMEM((TNODES + 16,), f32),
            pltpu.VMEM((128,), jnp.int32),
            pltpu.VMEM((128,), jnp.int32),
            pltpu.VMEM((128,), jnp.int32),
            pltpu.VMEM((128,), jnp.int32),
            pltpu.VMEM((128,), jnp.int32),
            pltpu.VMEM((128,), jnp.int32),
            pltpu.VMEM((128,), jnp.int32),
            pltpu.VMEM((128,), jnp.int32),
            pltpu.VMEM((128, HALF), f32),
            pltpu.VMEM((NCH, HALF), f32),
            pltpu.VMEM((NCH, HALF), f32),
            pltpu.VMEM((NCH, HALF), f32),
            pltpu.VMEM((ZR, HALF), f32),
            pltpu.VMEM_SHARED((NPAD, HALF), f32),
            pltpu.SemaphoreType.DMA,
            pltpu.SemaphoreType.DMA,
            pltpu.SemaphoreType.DMA,
            pltpu.SemaphoreType.DMA,
            pltpu.SemaphoreType.DMA,
        ],
    )
    ax, out, _ = fn(src_p, dst_p, x2)
    del ax
    return out


def _enc_kernel(t_ref, v_ref, tW_ref, tb_ref, tg_ref, tbb_ref, ta_ref,
                vW_ref, vb_ref, vg_ref, vbb_ref, va_ref, out_ref):
    inv = 1.0 / math.sqrt(1.0 + BN_EPS)
    ht = jnp.dot(t_ref[...], tW_ref[...], preferred_element_type=jnp.float32)
    ht = (ht + tb_ref[...]) * (inv * tg_ref[...]) + tbb_ref[...]
    ht = jnp.where(ht > 0, ht, ta_ref[...] * ht)
    hv = jnp.dot(v_ref[...], vW_ref[...], preferred_element_type=jnp.float32)
    hv = (hv + vb_ref[...]) * (inv * vg_ref[...]) + vbb_ref[...]
    hv = jnp.where(hv > 0, hv, va_ref[...] * hv)
    out_ref[...] = ht + hv


def _encode(item_t_feat, item_v_feat, t_W, t_b, t_bn_g, t_bn_b, t_a,
            v_W, v_b, v_bn_g, v_bn_b, v_a):
    R = 1000
    grid = (N_ITEMS // R,)
    row = lambda i: (i, 0)
    fix = lambda i: (0, 0)
    return pl.pallas_call(
        _enc_kernel,
        grid=grid,
        in_specs=[
            pl.BlockSpec((R, 384), row),
            pl.BlockSpec((R, 512), row),
            pl.BlockSpec((384, DIM), fix),
            pl.BlockSpec((1, DIM), fix),
            pl.BlockSpec((1, DIM), fix),
            pl.BlockSpec((1, DIM), fix),
            pl.BlockSpec((1, 1), fix),
            pl.BlockSpec((512, DIM), fix),
            pl.BlockSpec((1, DIM), fix),
            pl.BlockSpec((1, DIM), fix),
            pl.BlockSpec((1, DIM), fix),
            pl.BlockSpec((1, 1), fix),
        ],
        out_specs=pl.BlockSpec((R, DIM), row),
        out_shape=jax.ShapeDtypeStruct((N_ITEMS, DIM), jnp.float32),
    )(item_t_feat, item_v_feat,
      t_W, t_b.reshape(1, DIM), t_bn_g.reshape(1, DIM),
      t_bn_b.reshape(1, DIM), t_a.reshape(1, 1),
      v_W, v_b.reshape(1, DIM), v_bn_g.reshape(1, DIM),
      v_bn_b.reshape(1, DIM), v_a.reshape(1, 1))


def kernel(g, user_embeddings, item_v_feat, item_t_feat, t_W, t_b, t_bn_g,
           t_bn_b, t_a, v_W, v_b, v_bn_g, v_bn_b, v_a):
    enc = _encode(item_t_feat, item_v_feat, t_W, t_b, t_bn_g, t_bn_b, t_a,
                  v_W, v_b, v_bn_g, v_bn_b, v_a)
    x = jnp.concatenate([user_embeddings, enc], axis=0)

    x2 = jnp.zeros((NSC * NPAD, HALF), jnp.float32)
    x2 = x2.at[:N_NODES].set(x[:, :HALF])
    x2 = x2.at[NPAD:NPAD + N_NODES].set(x[:, HALF:])

    pad = jnp.full((EPAD - N_EDGES,), GARBAGE, jnp.int32)
    src_p = jnp.concatenate([g[0], pad])
    dst_p = jnp.concatenate([g[1], pad])

    out = _sc_diffuse(src_p, dst_p, x2, None)
    return jnp.concatenate([out[:N_NODES], out[NPAD:NPAD + N_NODES]], axis=1)


# speedup vs baseline: 18.4165x; 1.0021x over previous
import functools
import math

import jax
import jax.numpy as jnp
from jax import lax
from jax.experimental import pallas as pl
from jax.experimental.pallas import tpu as pltpu
from jax.experimental.pallas import tpu_sc as plsc

N_USERS = 25000
N_ITEMS = 25000
N_NODES = 50000
N_EDGES = 800000
DIM = 64
HALF = 32
KSTEPS = 4
ALPHA = 0.1
BETA = 0.9
BN_EPS = 1e-5
DENOM = BETA ** KSTEPS + ALPHA * sum(BETA ** i for i in range(KSTEPS))

NSC = 2
NT = 16
NPAD = 51200
TNODES = 3200
NCH = 128
ZR = 32
EPAD = 819200
ECH = 400
GARBAGE = 50000


def _sc_body(src_ref, dst_ref, x2_ref, ax2_ref_in, out_ref, gtab_ref,
             af, cbf, bdf, sidx, didx, sidx1, didx1, sidx2, didx2, sidx3,
             didx3, grow, xb, accb, gb, zb, acc, sem, sem1, isem, ssem0,
             ssem1):
    c = lax.axis_index("c")
    t = lax.axis_index("s")
    coff = c * NPAD
    ebase = t * (ECH * 128)
    gbase = coff + t * TNODES

    zeros16f = jnp.zeros((16,), jnp.float32)
    ones16 = jnp.full((16,), 1.0, jnp.float32)
    iota16 = lax.iota(jnp.int32, 16)
    zeros16i = jnp.zeros((16,), jnp.int32)

    @pl.loop(0, ZR)
    def _zero_zb(r):
        zb[r, pl.ds(0, 16)] = zeros16f
        zb[r, pl.ds(16, 16)] = zeros16f

    @pl.loop(0, 128)
    def _ones_grow(r):
        grow[r, pl.ds(0, 16)] = ones16
        grow[r, pl.ds(16, 16)] = ones16

    def _zero_acc_chunk(ch):
        for q in range(NCH // ZR):
            pltpu.sync_copy(
                zb, acc.at[pl.ds(t * TNODES + ch * NCH + q * ZR, ZR)])

    def _zero_acc():
        @pl.loop(0, TNODES // NCH)
        def _z(ch):
            _zero_acc_chunk(ch)

    SETS = ((sidx, didx), (sidx1, didx1), (sidx2, didx2), (sidx3, didx3))
    GROW = (grow, gb)
    GSEM = (sem, sem1)
    SSEM = (ssem0, ssem1)

    def _idx_start(edge_ref, i, s, both):
        off = ebase + i * 128
        if both:
            pltpu.async_copy(src_ref.at[c, pl.ds(off, 128)], SETS[s][0],
                             isem)
            pltpu.async_copy(dst_ref.at[pl.ds(off, 128)], SETS[s][1], isem)
        else:
            pltpu.async_copy(edge_ref.at[pl.ds(off, 128)], SETS[s][1], isem)

    def _idx_wait(edge_ref, i, s, both):
        off = ebase + i * 128
        if both:
            pltpu.make_async_copy(
                src_ref.at[c, pl.ds(off, 128)], SETS[s][0], isem).wait()
            pltpu.make_async_copy(
                dst_ref.at[pl.ds(off, 128)], SETS[s][1], isem).wait()
        else:
            pltpu.make_async_copy(
                edge_ref.at[pl.ds(off, 128)], SETS[s][1], isem).wait()

    def _scat_wait(g, did, ss):
        pltpu.make_async_copy(g, acc.at[did], ss).wait()

    def _deg_pass(edge_ref, dest):
        _zero_acc()
        plsc.subcore_barrier()

        _idx_start(edge_ref, 0, 0, False)
        _idx_start(edge_ref, 1, 1, False)

        @pl.loop(0, ECH // 4)
        def _scat(i4):
            for k in range(4):
                i = i4 * 4 + k
                did = SETS[k][1]
                ss = SSEM[k % 2]
                _idx_wait(edge_ref, i, k, False)
                if k < 2:
                    @pl.when(i4 > 0)
                    def _():
                        _scat_wait(grow, did, ss)
                else:
                    _scat_wait(grow, did, ss)
                pltpu.async_copy(grow, acc.at[did], ss, add=True)
                if k < 2:
                    _idx_start(edge_ref, i + 2, (k + 2) % 4, False)
                else:
                    @pl.when(i4 < ECH // 4 - 1)
                    def _():
                        _idx_start(edge_ref, i + 2, (k + 2) % 4, False)

        _scat_wait(grow, SETS[2][1], SSEM[0])
        _scat_wait(grow, SETS[3][1], SSEM[1])
        plsc.subcore_barrier()

        @pl.loop(0, TNODES // NCH)
        def _extract(ch):
            pltpu.sync_copy(acc.at[pl.ds(t * TNODES + ch * NCH, NCH)], accb)
            for k in range(NCH // 16):
                v = plsc.load_gather(accb, [iota16 + k * 16, zeros16i])
                dest[pl.ds(ch * NCH + k * 16, 16)] = v

    _deg_pass(src_ref.at[0], af)
    _deg_pass(dst_ref, bdf)

    def _rsqrt(d):
        i = plsc.bitcast(d, jnp.int32)
        y = plsc.bitcast(0x5F3759DF - (i >> 1), jnp.float32)
        for _ in range(3):
            y = y * (1.5 - 0.5 * d * y * y)
        return y

    @pl.loop(0, TNODES // 16)
    def _scales(i):
        sl = pl.ds(i * 16, 16)
        av = _rsqrt(jnp.maximum(af[sl], 1.0))
        bv = _rsqrt(jnp.maximum(bdf[sl], 1.0))
        af[sl] = av
        cbf[sl] = BETA * av * bv
        bdf[sl] = (BETA / DENOM) * bv

    @pl.loop(0, TNODES // NCH)
    def _g0(ch):
        base = gbase + ch * NCH
        pltpu.sync_copy(x2_ref.at[pl.ds(base, NCH)], xb)

        @pl.loop(0, NCH)
        def _rows(j):
            ln = ch * NCH + j
            av = af[pl.ds(ln, 16)][0]
            for f in range(2):
                sl = pl.ds(16 * f, 16)
                gb[j, sl] = av * xb[j, sl]

        pltpu.sync_copy(gb, gtab_ref.at[pl.ds(base, NCH)])
        pltpu.sync_copy(gb, ax2_ref_in.at[pl.ds(base, NCH)])
        _zero_acc_chunk(ch)

    plsc.subcore_barrier()

    @pl.loop(0, KSTEPS)
    def _step(it):
        _idx_start(src_ref, 0, 0, True)
        _idx_start(src_ref, 1, 1, True)

        @pl.loop(0, ECH // 4)
        def _edges(i4):
            for k in range(4):
                i = i4 * 4 + k
                sid, did = SETS[k]
                g = GROW[k % 2]
                _idx_wait(src_ref, i, k, True)
                if k < 2:
                    @pl.when(i4 > 0)
                    def _():
                        _scat_wait(g, did, SSEM[k % 2])
                else:
                    _scat_wait(g, did, SSEM[k % 2])
                pltpu.async_copy(gtab_ref.at[sid], g, GSEM[k % 2])
                if k < 2:
                    _idx_start(src_ref, i + 2, (k + 2) % 4, True)
                else:
                    @pl.when(i4 < ECH // 4 - 1)
                    def _():
                        _idx_start(src_ref, i + 2, (k + 2) % 4, True)
                pg = GROW[(k + 1) % 2]
                psid, pdid = SETS[(k + 3) % 4]

                def _finish_prev(pg=pg, psid=psid, pdid=pdid, k=k):
                    pltpu.make_async_copy(gtab_ref.at[psid], pg,
                                          GSEM[(k + 1) % 2]).wait()
                    pltpu.async_copy(pg, acc.at[pdid],
                                     SSEM[(k + 1) % 2], add=True)

                if k == 0:
                    pl.when(i4 > 0)(_finish_prev)
                else:
                    _finish_prev()

        pltpu.make_async_copy(gtab_ref.at[SETS[3][0]], GROW[1],
                              GSEM[1]).wait()
        pltpu.async_copy(GROW[1], acc.at[SETS[3][1]], SSEM[1], add=True)
        _scat_wait(GROW[0], SETS[2][1], SSEM[0])
        _scat_wait(GROW[1], SETS[3][1], SSEM[1])
        plsc.subcore_barrier()

        last = it == KSTEPS - 1

        @pl.loop(0, TNODES // NCH)
        def _nodes(ch):
            base = gbase + ch * NCH

            @pl.when(jnp.logical_not(last))
            def _():
                pltpu.sync_copy(ax2_ref_in.at[pl.ds(base, NCH)], xb)

            @pl.when(last)
            def _():
                pltpu.sync_copy(x2_ref.at[pl.ds(base, NCH)], xb)

            pltpu.sync_copy(acc.at[pl.ds(t * TNODES + ch * NCH, NCH)], accb)

            @pl.loop(0, NCH)
            def _rows(j):
                ln = ch * NCH + j
                cbv = cbf[pl.ds(ln, 16)][0]
                bdv = bdf[pl.ds(ln, 16)][0]
                for f in range(2):
                    sl = pl.ds(16 * f, 16)
                    un = ALPHA * xb[j, sl] + cbv * accb[j, sl]
                    ov = (ALPHA / DENOM) * xb[j, sl] + bdv * accb[j, sl]
                    gb[j, sl] = jnp.where(last, ov, un)

            @pl.when(jnp.logical_not(last))
            def _():
                pltpu.sync_copy(gb, gtab_ref.at[pl.ds(base, NCH)])

            @pl.when(last)
            def _():
                pltpu.sync_copy(gb, out_ref.at[pl.ds(base, NCH)])

            _zero_acc_chunk(ch)

        plsc.subcore_barrier()


def _sc_diffuse(src_p, dst_p, x2, ax_init):
    f32 = jnp.float32
    mesh = plsc.VectorSubcoreMesh(core_axis_name="c", subcore_axis_name="s")
    fn = pl.kernel(
        _sc_body,
        out_type=[
            jax.ShapeDtypeStruct((NSC * NPAD, HALF), f32),
            jax.ShapeDtypeStruct((NSC * NPAD, HALF), f32),
            jax.ShapeDtypeStruct((NSC * NPAD, HALF), f32),
        ],
        mesh=mesh,
        compiler_params=pltpu.CompilerParams(needs_layout_passes=False,
                                             use_tc_tiling_on_sc=False),
        scratch_types=[
            pltpu.VMEM((TNODES + 16,), f32),
            pltpu.VMEM((TNODES + 16,), f32),
            pltpu.VMEM((TNODES + 16,), f32),
            pltpu.VMEM((128,), jnp.int32),
            pltpu.VMEM((128,), jnp.int32),
            pltpu.VMEM((128,), jnp.int32),
            pltpu.VMEM((128,), jnp.int32),
            pltpu.VMEM((128,), jnp.int32),
            pltpu.VMEM((128,), jnp.int32),
            pltpu.VMEM((128,), jnp.int32),
            pltpu.VMEM((128,), jnp.int32),
            pltpu.VMEM((128, HALF), f32),
            pltpu.VMEM((NCH, HALF), f32),
            pltpu.VMEM((NCH, HALF), f32),
            pltpu.VMEM((NCH, HALF), f32),
            pltpu.VMEM((ZR, HALF), f32),
            pltpu.VMEM_SHARED((NPAD, HALF), f32),
            pltpu.SemaphoreType.DMA,
            pltpu.SemaphoreType.DMA,
            pltpu.SemaphoreType.DMA,
            pltpu.SemaphoreType.DMA,
            pltpu.SemaphoreType.DMA,
        ],
    )
    ax, out, _ = fn(src_p, dst_p, x2)
    del ax
    return out


def _enc_kernel(t_ref, v_ref, tW_ref, tb_ref, tg_ref, tbb_ref, ta_ref,
                vW_ref, vb_ref, vg_ref, vbb_ref, va_ref, out_ref):
    inv = 1.0 / math.sqrt(1.0 + BN_EPS)
    ht = jnp.dot(t_ref[...], tW_ref[...], preferred_element_type=jnp.float32)
    ht = (ht + tb_ref[...]) * (inv * tg_ref[...]) + tbb_ref[...]
    ht = jnp.where(ht > 0, ht, ta_ref[...] * ht)
    hv = jnp.dot(v_ref[...], vW_ref[...], preferred_element_type=jnp.float32)
    hv = (hv + vb_ref[...]) * (inv * vg_ref[...]) + vbb_ref[...]
    hv = jnp.where(hv > 0, hv, va_ref[...] * hv)
    out_ref[...] = ht + hv


def _encode(item_t_feat, item_v_feat, t_W, t_b, t_bn_g, t_bn_b, t_a,
            v_W, v_b, v_bn_g, v_bn_b, v_a):
    R = 1000
    grid = (N_ITEMS // R,)
    row = lambda i: (i, 0)
    fix = lambda i: (0, 0)
    return pl.pallas_call(
        _enc_kernel,
        grid=grid,
        in_specs=[
            pl.BlockSpec((R, 384), row),
            pl.BlockSpec((R, 512), row),
            pl.BlockSpec((384, DIM), fix),
            pl.BlockSpec((1, DIM), fix),
            pl.BlockSpec((1, DIM), fix),
            pl.BlockSpec((1, DIM), fix),
            pl.BlockSpec((1, 1), fix),
            pl.BlockSpec((512, DIM), fix),
            pl.BlockSpec((1, DIM), fix),
            pl.BlockSpec((1, DIM), fix),
            pl.BlockSpec((1, DIM), fix),
            pl.BlockSpec((1, 1), fix),
        ],
        out_specs=pl.BlockSpec((R, DIM), row),
        out_shape=jax.ShapeDtypeStruct((N_ITEMS, DIM), jnp.float32),
    )(item_t_feat, item_v_feat,
      t_W, t_b.reshape(1, DIM), t_bn_g.reshape(1, DIM),
      t_bn_b.reshape(1, DIM), t_a.reshape(1, 1),
      v_W, v_b.reshape(1, DIM), v_bn_g.reshape(1, DIM),
      v_bn_b.reshape(1, DIM), v_a.reshape(1, 1))


def kernel(g, user_embeddings, item_v_feat, item_t_feat, t_W, t_b, t_bn_g,
           t_bn_b, t_a, v_W, v_b, v_bn_g, v_bn_b, v_a):
    enc = _encode(item_t_feat, item_v_feat, t_W, t_b, t_bn_g, t_bn_b, t_a,
                  v_W, v_b, v_bn_g, v_bn_b, v_a)
    x = jnp.concatenate([user_embeddings, enc], axis=0)

    x2 = jnp.zeros((NSC * NPAD, HALF), jnp.float32)
    x2 = x2.at[:N_NODES].set(x[:, :HALF])
    x2 = x2.at[NPAD:NPAD + N_NODES].set(x[:, HALF:])

    pad = jnp.full((EPAD - N_EDGES,), GARBAGE, jnp.int32)
    src_p = jnp.concatenate([g[0], pad])
    dst_p = jnp.concatenate([g[1], pad])
    src_p = jnp.stack([src_p, src_p + NPAD])

    out = _sc_diffuse(src_p, dst_p, x2, None)
    return jnp.concatenate([out[:N_NODES], out[NPAD:NPAD + N_NODES]], axis=1)


# speedup vs baseline: 19.6689x; 1.0680x over previous
import functools
import math

import jax
import jax.numpy as jnp
from jax import lax
from jax.experimental import pallas as pl
from jax.experimental.pallas import tpu as pltpu
from jax.experimental.pallas import tpu_sc as plsc

N_USERS = 25000
N_ITEMS = 25000
N_NODES = 50000
N_EDGES = 800000
DIM = 64
HALF = 32
KSTEPS = 4
ALPHA = 0.1
BETA = 0.9
BN_EPS = 1e-5
DENOM = BETA ** KSTEPS + ALPHA * sum(BETA ** i for i in range(KSTEPS))

NSC = 2
NT = 16
NPAD = 51200
TNODES = 3200
NCH = 128
ZR = 16
EPAD = 819200
ECH = 400
GARBAGE = 50000


def _sc_body(src_ref, dst_ref, x2_ref, ax2_ref_in, out_ref, gtab_ref,
             af, cbf, bdf, sidx, didx, sidx1, didx1, sidx2, didx2, sidx3,
             didx3, sidx4, didx4, sidx5, didx5, sidx6, didx6, sidx7, didx7,
             grow, xb, accb, gb, zb, acc, sem, sem1, sem2, sem3, isem,
             ssem0, ssem1, ssem2, ssem3):
    c = lax.axis_index("c")
    t = lax.axis_index("s")
    coff = c * NPAD
    ebase = t * (ECH * 128)
    gbase = coff + t * TNODES

    zeros16f = jnp.zeros((16,), jnp.float32)
    ones16 = jnp.full((16,), 1.0, jnp.float32)
    iota16 = lax.iota(jnp.int32, 16)
    zeros16i = jnp.zeros((16,), jnp.int32)

    @pl.loop(0, ZR)
    def _zero_zb(r):
        zb[r, pl.ds(0, 16)] = zeros16f
        zb[r, pl.ds(16, 16)] = zeros16f

    @pl.loop(0, 128)
    def _ones_grow(r):
        grow[r, pl.ds(0, 16)] = ones16
        grow[r, pl.ds(16, 16)] = ones16

    def _zero_acc_chunk(ch):
        for q in range(NCH // ZR):
            pltpu.sync_copy(
                zb, acc.at[pl.ds(t * TNODES + ch * NCH + q * ZR, ZR)])

    def _zero_acc():
        @pl.loop(0, TNODES // NCH)
        def _z(ch):
            _zero_acc_chunk(ch)

    SETS = ((sidx, didx), (sidx1, didx1), (sidx2, didx2), (sidx3, didx3),
            (sidx4, didx4), (sidx5, didx5), (sidx6, didx6), (sidx7, didx7))
    GROW = (grow, gb, xb, accb)
    GSEM = (sem, sem1, sem2, sem3)
    SSEM = (ssem0, ssem1, ssem2, ssem3)

    def _idx_start(edge_ref, i, s, both):
        off = ebase + i * 128
        if both:
            pltpu.async_copy(src_ref.at[c, pl.ds(off, 128)], SETS[s][0],
                             isem)
            pltpu.async_copy(dst_ref.at[pl.ds(off, 128)], SETS[s][1], isem)
        else:
            pltpu.async_copy(edge_ref.at[pl.ds(off, 128)], SETS[s][1], isem)

    def _idx_wait(edge_ref, i, s, both):
        off = ebase + i * 128
        if both:
            pltpu.make_async_copy(
                src_ref.at[c, pl.ds(off, 128)], SETS[s][0], isem).wait()
            pltpu.make_async_copy(
                dst_ref.at[pl.ds(off, 128)], SETS[s][1], isem).wait()
        else:
            pltpu.make_async_copy(
                edge_ref.at[pl.ds(off, 128)], SETS[s][1], isem).wait()

    def _scat_wait(g, did, ss):
        pltpu.make_async_copy(g, acc.at[did], ss).wait()

    def _deg_pass(edge_ref, dest):
        _zero_acc()
        plsc.subcore_barrier()

        _idx_start(edge_ref, 0, 0, False)
        _idx_start(edge_ref, 1, 1, False)

        @pl.loop(0, ECH // 4)
        def _scat(i4):
            for k in range(4):
                i = i4 * 4 + k
                did = SETS[k][1]
                ss = SSEM[k % 2]
                _idx_wait(edge_ref, i, k, False)
                if k < 2:
                    @pl.when(i4 > 0)
                    def _():
                        _scat_wait(grow, did, ss)
                else:
                    _scat_wait(grow, did, ss)
                pltpu.async_copy(grow, acc.at[did], ss, add=True)
                if k < 2:
                    _idx_start(edge_ref, i + 2, (k + 2) % 4, False)
                else:
                    @pl.when(i4 < ECH // 4 - 1)
                    def _():
                        _idx_start(edge_ref, i + 2, (k + 2) % 4, False)

        _scat_wait(grow, SETS[2][1], SSEM[0])
        _scat_wait(grow, SETS[3][1], SSEM[1])
        plsc.subcore_barrier()

        @pl.loop(0, TNODES // NCH)
        def _extract(ch):
            pltpu.sync_copy(acc.at[pl.ds(t * TNODES + ch * NCH, NCH)], accb)
            for k in range(NCH // 16):
                v = plsc.load_gather(accb, [iota16 + k * 16, zeros16i])
                dest[pl.ds(ch * NCH + k * 16, 16)] = v

    _deg_pass(src_ref.at[0], af)
    _deg_pass(dst_ref, bdf)

    def _rsqrt(d):
        i = plsc.bitcast(d, jnp.int32)
        y = plsc.bitcast(0x5F3759DF - (i >> 1), jnp.float32)
        for _ in range(3):
            y = y * (1.5 - 0.5 * d * y * y)
        return y

    @pl.loop(0, TNODES // 16)
    def _scales(i):
        sl = pl.ds(i * 16, 16)
        av = _rsqrt(jnp.maximum(af[sl], 1.0))
        bv = _rsqrt(jnp.maximum(bdf[sl], 1.0))
        af[sl] = av
        cbf[sl] = BETA * av * bv
        bdf[sl] = (BETA / DENOM) * bv

    @pl.loop(0, TNODES // NCH)
    def _g0(ch):
        base = gbase + ch * NCH
        pltpu.sync_copy(x2_ref.at[pl.ds(base, NCH)], xb)

        @pl.loop(0, NCH)
        def _rows(j):
            ln = ch * NCH + j
            av = af[pl.ds(ln, 16)][0]
            for f in range(2):
                sl = pl.ds(16 * f, 16)
                gb[j, sl] = av * xb[j, sl]

        pltpu.sync_copy(gb, gtab_ref.at[pl.ds(base, NCH)])
        pltpu.sync_copy(gb, ax2_ref_in.at[pl.ds(base, NCH)])
        _zero_acc_chunk(ch)

    plsc.subcore_barrier()

    @pl.loop(0, KSTEPS)
    def _step(it):
        for w in range(4):
            _idx_start(src_ref, w, w, True)

        @pl.loop(0, ECH // 8)
        def _edges(i8):
            for k in range(8):
                i = i8 * 8 + k
                b = k % 4
                sid, did = SETS[k]
                g = GROW[b]
                _idx_wait(src_ref, i, k, True)

                def _free(g=g, did=did, b=b):
                    _scat_wait(g, did, SSEM[b])

                if k < 4:
                    pl.when(i8 > 0)(_free)
                else:
                    _free()
                pltpu.async_copy(gtab_ref.at[sid], g, GSEM[b])

                def _prefetch(i=i, k=k):
                    _idx_start(src_ref, i + 4, (k + 4) % 8, True)

                if k < 4:
                    _prefetch()
                else:
                    pl.when(i8 < ECH // 8 - 1)(_prefetch)

                pb = (b + 2) % 4
                psid, pdid = SETS[(k + 6) % 8]

                def _finish(pb=pb, psid=psid, pdid=pdid):
                    pltpu.make_async_copy(gtab_ref.at[psid], GROW[pb],
                                          GSEM[pb]).wait()
                    pltpu.async_copy(GROW[pb], acc.at[pdid],
                                     SSEM[pb], add=True)

                if k < 2:
                    pl.when(i8 > 0)(_finish)
                else:
                    _finish()

        for i in (ECH - 2, ECH - 1):
            b = i % 4
            psid, pdid = SETS[i % 8]
            pltpu.make_async_copy(gtab_ref.at[psid], GROW[b],
                                  GSEM[b]).wait()
            pltpu.async_copy(GROW[b], acc.at[pdid], SSEM[b], add=True)
        for i in (ECH - 4, ECH - 3, ECH - 2, ECH - 1):
            _scat_wait(GROW[i % 4], SETS[i % 8][1], SSEM[i % 4])
        plsc.subcore_barrier()

        last = it == KSTEPS - 1

        @pl.loop(0, TNODES // NCH)
        def _nodes(ch):
            base = gbase + ch * NCH

            @pl.when(jnp.logical_not(last))
            def _():
                pltpu.sync_copy(ax2_ref_in.at[pl.ds(base, NCH)], xb)

            @pl.when(last)
            def _():
                pltpu.sync_copy(x2_ref.at[pl.ds(base, NCH)], xb)

            pltpu.sync_copy(acc.at[pl.ds(t * TNODES + ch * NCH, NCH)], accb)

            @pl.loop(0, NCH)
            def _rows(j):
                ln = ch * NCH + j
                cbv = cbf[pl.ds(ln, 16)][0]
                bdv = bdf[pl.ds(ln, 16)][0]
                for f in range(2):
                    sl = pl.ds(16 * f, 16)
                    un = ALPHA * xb[j, sl] + cbv * accb[j, sl]
                    ov = (ALPHA / DENOM) * xb[j, sl] + bdv * accb[j, sl]
                    gb[j, sl] = jnp.where(last, ov, un)

            @pl.when(jnp.logical_not(last))
            def _():
                pltpu.sync_copy(gb, gtab_ref.at[pl.ds(base, NCH)])

            @pl.when(last)
            def _():
                pltpu.sync_copy(gb, out_ref.at[pl.ds(base, NCH)])

            _zero_acc_chunk(ch)

        plsc.subcore_barrier()


def _sc_diffuse(src_p, dst_p, x2, ax_init):
    f32 = jnp.float32
    mesh = plsc.VectorSubcoreMesh(core_axis_name="c", subcore_axis_name="s")
    fn = pl.kernel(
        _sc_body,
        out_type=[
            jax.ShapeDtypeStruct((NSC * NPAD, HALF), f32),
            jax.ShapeDtypeStruct((NSC * NPAD, HALF), f32),
            jax.ShapeDtypeStruct((NSC * NPAD, HALF), f32),
        ],
        mesh=mesh,
        compiler_params=pltpu.CompilerParams(needs_layout_passes=False,
                                             use_tc_tiling_on_sc=False),
        scratch_types=[
            pltpu.VMEM((TNODES + 16,), f32),
            pltpu.VMEM((TNODES + 16,), f32),
            pltpu.VMEM((TNODES + 16,), f32),
            pltpu.VMEM((128,), jnp.int32),
            pltpu.VMEM((128,), jnp.int32),
            pltpu.VMEM((128,), jnp.int32),
            pltpu.VMEM((128,), jnp.int32),
            pltpu.VMEM((128,), jnp.int32),
            pltpu.VMEM((128,), jnp.int32),
            pltpu.VMEM((128,), jnp.int32),
            pltpu.VMEM((128,), jnp.int32),
            pltpu.VMEM((128,), jnp.int32),
            pltpu.VMEM((128,), jnp.int32),
            pltpu.VMEM((128,), jnp.int32),
            pltpu.VMEM((128,), jnp.int32),
            pltpu.VMEM((128,), jnp.int32),
            pltpu.VMEM((128,), jnp.int32),
            pltpu.VMEM((128,), jnp.int32),
            pltpu.VMEM((128,), jnp.int32),
            pltpu.VMEM((128, HALF), f32),
            pltpu.VMEM((NCH, HALF), f32),
            pltpu.VMEM((NCH, HALF), f32),
            pltpu.VMEM((NCH, HALF), f32),
            pltpu.VMEM((ZR, HALF), f32),
            pltpu.VMEM_SHARED((NPAD, HALF), f32),
            pltpu.SemaphoreType.DMA,
            pltpu.SemaphoreType.DMA,
            pltpu.SemaphoreType.DMA,
            pltpu.SemaphoreType.DMA,
            pltpu.SemaphoreType.DMA,
            pltpu.SemaphoreType.DMA,
            pltpu.SemaphoreType.DMA,
            pltpu.SemaphoreType.DMA,
            pltpu.SemaphoreType.DMA,
        ],
    )
    ax, out, _ = fn(src_p, dst_p, x2)
    del ax
    return out


def _enc_kernel(t_ref, v_ref, tW_ref, tb_ref, tg_ref, tbb_ref, ta_ref,
                vW_ref, vb_ref, vg_ref, vbb_ref, va_ref, out_ref):
    inv = 1.0 / math.sqrt(1.0 + BN_EPS)
    ht = jnp.dot(t_ref[...], tW_ref[...], preferred_element_type=jnp.float32)
    ht = (ht + tb_ref[...]) * (inv * tg_ref[...]) + tbb_ref[...]
    ht = jnp.where(ht > 0, ht, ta_ref[...] * ht)
    hv = jnp.dot(v_ref[...], vW_ref[...], preferred_element_type=jnp.float32)
    hv = (hv + vb_ref[...]) * (inv * vg_ref[...]) + vbb_ref[...]
    hv = jnp.where(hv > 0, hv, va_ref[...] * hv)
    out_ref[...] = ht + hv


def _encode(item_t_feat, item_v_feat, t_W, t_b, t_bn_g, t_bn_b, t_a,
            v_W, v_b, v_bn_g, v_bn_b, v_a):
    R = 1000
    grid = (N_ITEMS // R,)
    row = lambda i: (i, 0)
    fix = lambda i: (0, 0)
    return pl.pallas_call(
        _enc_kernel,
        grid=grid,
        in_specs=[
            pl.BlockSpec((R, 384), row),
            pl.BlockSpec((R, 512), row),
            pl.BlockSpec((384, DIM), fix),
            pl.BlockSpec((1, DIM), fix),
            pl.BlockSpec((1, DIM), fix),
            pl.BlockSpec((1, DIM), fix),
            pl.BlockSpec((1, 1), fix),
            pl.BlockSpec((512, DIM), fix),
            pl.BlockSpec((1, DIM), fix),
            pl.BlockSpec((1, DIM), fix),
            pl.BlockSpec((1, DIM), fix),
            pl.BlockSpec((1, 1), fix),
        ],
        out_specs=pl.BlockSpec((R, DIM), row),
        out_shape=jax.ShapeDtypeStruct((N_ITEMS, DIM), jnp.float32),
    )(item_t_feat, item_v_feat,
      t_W, t_b.reshape(1, DIM), t_bn_g.reshape(1, DIM),
      t_bn_b.reshape(1, DIM), t_a.reshape(1, 1),
      v_W, v_b.reshape(1, DIM), v_bn_g.reshape(1, DIM),
      v_bn_b.reshape(1, DIM), v_a.reshape(1, 1))


def kernel(g, user_embeddings, item_v_feat, item_t_feat, t_W, t_b, t_bn_g,
           t_bn_b, t_a, v_W, v_b, v_bn_g, v_bn_b, v_a):
    enc = _encode(item_t_feat, item_v_feat, t_W, t_b, t_bn_g, t_bn_b, t_a,
                  v_W, v_b, v_bn_g, v_bn_b, v_a)
    x = jnp.concatenate([user_embeddings, enc], axis=0)

    x2 = jnp.zeros((NSC * NPAD, HALF), jnp.float32)
    x2 = x2.at[:N_NODES].set(x[:, :HALF])
    x2 = x2.at[NPAD:NPAD + N_NODES].set(x[:, HALF:])

    pad = jnp.full((EPAD - N_EDGES,), GARBAGE, jnp.int32)
    src_p = jnp.concatenate([g[0], pad])
    dst_p = jnp.concatenate([g[1], pad])
    src_p = jnp.stack([src_p, src_p + NPAD])

    out = _sc_diffuse(src_p, dst_p, x2, None)
    return jnp.concatenate([out[:N_NODES], out[NPAD:NPAD + N_NODES]], axis=1)


# speedup vs baseline: 20.1159x; 1.0227x over previous
import functools
import math

import jax
import jax.numpy as jnp
from jax import lax
from jax.experimental import pallas as pl
from jax.experimental.pallas import tpu as pltpu
from jax.experimental.pallas import tpu_sc as plsc

N_USERS = 25000
N_ITEMS = 25000
N_NODES = 50000
N_EDGES = 800000
DIM = 64
HALF = 32
KSTEPS = 4
ALPHA = 0.1
BETA = 0.9
BN_EPS = 1e-5
DENOM = BETA ** KSTEPS + ALPHA * sum(BETA ** i for i in range(KSTEPS))

NSC = 2
NT = 16
NPAD = 51200
TNODES = 3200
NCH = 128
ZR = 16
EPAD = 819200
ECH = 400
GARBAGE = 50000


def _sc_body(src_ref, dst_ref, x2_ref, ax2_ref_in, out_ref, gtab_ref,
             af, cbf, bdf, sidx, didx, sidx1, didx1, sidx2, didx2, sidx3,
             didx3, sidx4, didx4, sidx5, didx5, sidx6, didx6, sidx7, didx7,
             grow, xb, accb, gb, zb, acc, sem, sem1, sem2, sem3, isem,
             ssem0, ssem1, ssem2, ssem3):
    c = lax.axis_index("c")
    t = lax.axis_index("s")
    coff = c * NPAD
    ebase = t * (ECH * 128)
    gbase = coff + t * TNODES

    zeros16f = jnp.zeros((16,), jnp.float32)
    ones16 = jnp.full((16,), 1.0, jnp.float32)
    iota16 = lax.iota(jnp.int32, 16)
    zeros16i = jnp.zeros((16,), jnp.int32)

    @pl.loop(0, ZR)
    def _zero_zb(r):
        zb[r, pl.ds(0, 16)] = zeros16f
        zb[r, pl.ds(16, 16)] = zeros16f

    @pl.loop(0, 128)
    def _ones_grow(r):
        grow[r, pl.ds(0, 16)] = ones16
        grow[r, pl.ds(16, 16)] = ones16

    def _zero_acc_chunk(ch):
        for q in range(NCH // ZR):
            pltpu.sync_copy(
                zb, acc.at[pl.ds(t * TNODES + ch * NCH + q * ZR, ZR)])

    def _zero_chunk_start(ch):
        for q in range(NCH // ZR):
            pltpu.async_copy(
                zb, acc.at[pl.ds(t * TNODES + ch * NCH + q * ZR, ZR)], isem)

    def _zero_chunk_drain(ch):
        for q in range(NCH // ZR):
            pltpu.make_async_copy(
                zb, acc.at[pl.ds(t * TNODES + ch * NCH + q * ZR, ZR)],
                isem).wait()

    def _zero_acc():
        @pl.loop(0, TNODES // NCH)
        def _z(ch):
            _zero_acc_chunk(ch)

    SETS = ((sidx, didx), (sidx1, didx1), (sidx2, didx2), (sidx3, didx3),
            (sidx4, didx4), (sidx5, didx5), (sidx6, didx6), (sidx7, didx7))
    GROW = (grow, gb, xb, accb)
    GSEM = (sem, sem1, sem2, sem3)
    SSEM = (ssem0, ssem1, ssem2, ssem3)

    def _idx_start(edge_ref, i, s, both):
        off = ebase + i * 128
        if both:
            pltpu.async_copy(src_ref.at[c, pl.ds(off, 128)], SETS[s][0],
                             isem)
            pltpu.async_copy(dst_ref.at[pl.ds(off, 128)], SETS[s][1], isem)
        else:
            pltpu.async_copy(edge_ref.at[pl.ds(off, 128)], SETS[s][1], isem)

    def _idx_wait(edge_ref, i, s, both):
        off = ebase + i * 128
        if both:
            pltpu.make_async_copy(
                src_ref.at[c, pl.ds(off, 128)], SETS[s][0], isem).wait()
            pltpu.make_async_copy(
                dst_ref.at[pl.ds(off, 128)], SETS[s][1], isem).wait()
        else:
            pltpu.make_async_copy(
                edge_ref.at[pl.ds(off, 128)], SETS[s][1], isem).wait()

    def _scat_wait(g, did, ss):
        pltpu.make_async_copy(g, acc.at[did], ss).wait()

    def _deg_pass(edge_ref, dest):
        _zero_acc()
        plsc.subcore_barrier()

        _idx_start(edge_ref, 0, 0, False)
        _idx_start(edge_ref, 1, 1, False)

        @pl.loop(0, ECH // 4)
        def _scat(i4):
            for k in range(4):
                i = i4 * 4 + k
                did = SETS[k][1]
                ss = SSEM[k % 2]
                _idx_wait(edge_ref, i, k, False)
                if k < 2:
                    @pl.when(i4 > 0)
                    def _():
                        _scat_wait(grow, did, ss)
                else:
                    _scat_wait(grow, did, ss)
                pltpu.async_copy(grow, acc.at[did], ss, add=True)
                if k < 2:
                    _idx_start(edge_ref, i + 2, (k + 2) % 4, False)
                else:
                    @pl.when(i4 < ECH // 4 - 1)
                    def _():
                        _idx_start(edge_ref, i + 2, (k + 2) % 4, False)

        _scat_wait(grow, SETS[2][1], SSEM[0])
        _scat_wait(grow, SETS[3][1], SSEM[1])
        plsc.subcore_barrier()

        @pl.loop(0, TNODES // NCH)
        def _extract(ch):
            pltpu.sync_copy(acc.at[pl.ds(t * TNODES + ch * NCH, NCH)], accb)
            for k in range(NCH // 16):
                v = plsc.load_gather(accb, [iota16 + k * 16, zeros16i])
                dest[pl.ds(ch * NCH + k * 16, 16)] = v

    _deg_pass(src_ref.at[0], af)
    _deg_pass(dst_ref, bdf)

    def _rsqrt(d):
        i = plsc.bitcast(d, jnp.int32)
        y = plsc.bitcast(0x5F3759DF - (i >> 1), jnp.float32)
        for _ in range(3):
            y = y * (1.5 - 0.5 * d * y * y)
        return y

    @pl.loop(0, TNODES // 16)
    def _scales(i):
        sl = pl.ds(i * 16, 16)
        av = _rsqrt(jnp.maximum(af[sl], 1.0))
        bv = _rsqrt(jnp.maximum(bdf[sl], 1.0))
        af[sl] = av
        cbf[sl] = BETA * av * bv
        bdf[sl] = (BETA / DENOM) * bv

    @pl.loop(0, TNODES // NCH)
    def _g0(ch):
        base = gbase + ch * NCH
        pltpu.sync_copy(x2_ref.at[pl.ds(base, NCH)], xb)

        @pl.loop(0, NCH)
        def _rows(j):
            ln = ch * NCH + j
            av = af[pl.ds(ln, 16)][0]
            for f in range(2):
                sl = pl.ds(16 * f, 16)
                gb[j, sl] = av * xb[j, sl]

        pltpu.sync_copy(gb, gtab_ref.at[pl.ds(base, NCH)])
        pltpu.sync_copy(gb, ax2_ref_in.at[pl.ds(base, NCH)])

        @pl.when(ch > 0)
        def _():
            _zero_chunk_drain(ch - 1)

        _zero_chunk_start(ch)

    _zero_chunk_drain(TNODES // NCH - 1)
    plsc.subcore_barrier()

    @pl.loop(0, KSTEPS)
    def _step(it):
        for w in range(4):
            _idx_start(src_ref, w, w, True)

        @pl.loop(0, ECH // 8)
        def _edges(i8):
            for k in range(8):
                i = i8 * 8 + k
                b = k % 4
                sid, did = SETS[k]
                g = GROW[b]
                _idx_wait(src_ref, i, k, True)

                def _free(g=g, did=did, b=b):
                    _scat_wait(g, did, SSEM[b])

                if k < 4:
                    pl.when(i8 > 0)(_free)
                else:
                    _free()
                pltpu.async_copy(gtab_ref.at[sid], g, GSEM[b])

                def _prefetch(i=i, k=k):
                    _idx_start(src_ref, i + 4, (k + 4) % 8, True)

                if k < 4:
                    _prefetch()
                else:
                    pl.when(i8 < ECH // 8 - 1)(_prefetch)

                pb = (b + 2) % 4
                psid, pdid = SETS[(k + 6) % 8]

                def _finish(pb=pb, psid=psid, pdid=pdid):
                    pltpu.make_async_copy(gtab_ref.at[psid], GROW[pb],
                                          GSEM[pb]).wait()
                    pltpu.async_copy(GROW[pb], acc.at[pdid],
                                     SSEM[pb], add=True)

                if k < 2:
                    pl.when(i8 > 0)(_finish)
                else:
                    _finish()

        for i in (ECH - 2, ECH - 1):
            b = i % 4
            psid, pdid = SETS[i % 8]
            pltpu.make_async_copy(gtab_ref.at[psid], GROW[b],
                                  GSEM[b]).wait()
            pltpu.async_copy(GROW[b], acc.at[pdid], SSEM[b], add=True)
        for i in (ECH - 4, ECH - 3, ECH - 2, ECH - 1):
            _scat_wait(GROW[i % 4], SETS[i % 8][1], SSEM[i % 4])
        plsc.subcore_barrier()

        last = it == KSTEPS - 1

        @pl.loop(0, TNODES // NCH)
        def _nodes(ch):
            base = gbase + ch * NCH

            @pl.when(jnp.logical_not(last))
            def _():
                pltpu.sync_copy(ax2_ref_in.at[pl.ds(base, NCH)], xb)

            @pl.when(last)
            def _():
                pltpu.sync_copy(x2_ref.at[pl.ds(base, NCH)], xb)

            pltpu.sync_copy(acc.at[pl.ds(t * TNODES + ch * NCH, NCH)], accb)

            @pl.loop(0, NCH)
            def _rows(j):
                ln = ch * NCH + j
                cbv = cbf[pl.ds(ln, 16)][0]
                bdv = bdf[pl.ds(ln, 16)][0]
                for f in range(2):
                    sl = pl.ds(16 * f, 16)
                    un = ALPHA * xb[j, sl] + cbv * accb[j, sl]
                    ov = (ALPHA / DENOM) * xb[j, sl] + bdv * accb[j, sl]
                    gb[j, sl] = jnp.where(last, ov, un)

            @pl.when(jnp.logical_not(last))
            def _():
                pltpu.sync_copy(gb, gtab_ref.at[pl.ds(base, NCH)])

            @pl.when(last)
            def _():
                pltpu.sync_copy(gb, out_ref.at[pl.ds(base, NCH)])

            @pl.when(ch > 0)
            def _():
                _zero_chunk_drain(ch - 1)

            _zero_chunk_start(ch)

        _zero_chunk_drain(TNODES // NCH - 1)
        plsc.subcore_barrier()


def _sc_diffuse(src_p, dst_p, x2, ax_init):
    f32 = jnp.float32
    mesh = plsc.VectorSubcoreMesh(core_axis_name="c", subcore_axis_name="s")
    fn = pl.kernel(
        _sc_body,
        out_type=[
            jax.ShapeDtypeStruct((NSC * NPAD, HALF), f32),
            jax.ShapeDtypeStruct((NSC * NPAD, HALF), f32),
            jax.ShapeDtypeStruct((NSC * NPAD, HALF), f32),
        ],
        mesh=mesh,
        compiler_params=pltpu.CompilerParams(needs_layout_passes=False,
                                             use_tc_tiling_on_sc=False),
        scratch_types=[
            pltpu.VMEM((TNODES + 16,), f32),
            pltpu.VMEM((TNODES + 16,), f32),
            pltpu.VMEM((TNODES + 16,), f32),
            pltpu.VMEM((128,), jnp.int32),
            pltpu.VMEM((128,), jnp.int32),
            pltpu.VMEM((128,), jnp.int32),
            pltpu.VMEM((128,), jnp.int32),
            pltpu.VMEM((128,), jnp.int32),
            pltpu.VMEM((128,), jnp.int32),
            pltpu.VMEM((128,), jnp.int32),
            pltpu.VMEM((128,), jnp.int32),
            pltpu.VMEM((128,), jnp.int32),
            pltpu.VMEM((128,), jnp.int32),
            pltpu.VMEM((128,), jnp.int32),
            pltpu.VMEM((128,), jnp.int32),
            pltpu.VMEM((128,), jnp.int32),
            pltpu.VMEM((128,), jnp.int32),
            pltpu.VMEM((128,), jnp.int32),
            pltpu.VMEM((128,), jnp.int32),
            pltpu.VMEM((128, HALF), f32),
            pltpu.VMEM((NCH, HALF), f32),
            pltpu.VMEM((NCH, HALF), f32),
            pltpu.VMEM((NCH, HALF), f32),
            pltpu.VMEM((ZR, HALF), f32),
            pltpu.VMEM_SHARED((NPAD, HALF), f32),
            pltpu.SemaphoreType.DMA,
            pltpu.SemaphoreType.DMA,
            pltpu.SemaphoreType.DMA,
            pltpu.SemaphoreType.DMA,
            pltpu.SemaphoreType.DMA,
            pltpu.SemaphoreType.DMA,
            pltpu.SemaphoreType.DMA,
            pltpu.SemaphoreType.DMA,
            pltpu.SemaphoreType.DMA,
        ],
    )
    ax, out, _ = fn(src_p, dst_p, x2)
    del ax
    return out


def _enc_kernel(t_ref, v_ref, tW_ref, tb_ref, tg_ref, tbb_ref, ta_ref,
                vW_ref, vb_ref, vg_ref, vbb_ref, va_ref, out_ref):
    inv = 1.0 / math.sqrt(1.0 + BN_EPS)
    ht = jnp.dot(t_ref[...], tW_ref[...], preferred_element_type=jnp.float32)
    ht = (ht + tb_ref[...]) * (inv * tg_ref[...]) + tbb_ref[...]
    ht = jnp.where(ht > 0, ht, ta_ref[...] * ht)
    hv = jnp.dot(v_ref[...], vW_ref[...], preferred_element_type=jnp.float32)
    hv = (hv + vb_ref[...]) * (inv * vg_ref[...]) + vbb_ref[...]
    hv = jnp.where(hv > 0, hv, va_ref[...] * hv)
    out_ref[...] = ht + hv


def _encode(item_t_feat, item_v_feat, t_W, t_b, t_bn_g, t_bn_b, t_a,
            v_W, v_b, v_bn_g, v_bn_b, v_a):
    R = 1000
    grid = (N_ITEMS // R,)
    row = lambda i: (i, 0)
    fix = lambda i: (0, 0)
    return pl.pallas_call(
        _enc_kernel,
        grid=grid,
        in_specs=[
            pl.BlockSpec((R, 384), row),
            pl.BlockSpec((R, 512), row),
            pl.BlockSpec((384, DIM), fix),
            pl.BlockSpec((1, DIM), fix),
            pl.BlockSpec((1, DIM), fix),
            pl.BlockSpec((1, DIM), fix),
            pl.BlockSpec((1, 1), fix),
            pl.BlockSpec((512, DIM), fix),
            pl.BlockSpec((1, DIM), fix),
            pl.BlockSpec((1, DIM), fix),
            pl.BlockSpec((1, DIM), fix),
            pl.BlockSpec((1, 1), fix),
        ],
        out_specs=pl.BlockSpec((R, DIM), row),
        out_shape=jax.ShapeDtypeStruct((N_ITEMS, DIM), jnp.float32),
    )(item_t_feat, item_v_feat,
      t_W, t_b.reshape(1, DIM), t_bn_g.reshape(1, DIM),
      t_bn_b.reshape(1, DIM), t_a.reshape(1, 1),
      v_W, v_b.reshape(1, DIM), v_bn_g.reshape(1, DIM),
      v_bn_b.reshape(1, DIM), v_a.reshape(1, 1))


def kernel(g, user_embeddings, item_v_feat, item_t_feat, t_W, t_b, t_bn_g,
           t_bn_b, t_a, v_W, v_b, v_bn_g, v_bn_b, v_a):
    enc = _encode(item_t_feat, item_v_feat, t_W, t_b, t_bn_g, t_bn_b, t_a,
                  v_W, v_b, v_bn_g, v_bn_b, v_a)
    x = jnp.concatenate([user_embeddings, enc], axis=0)

    x2 = jnp.zeros((NSC * NPAD, HALF), jnp.float32)
    x2 = x2.at[:N_NODES].set(x[:, :HALF])
    x2 = x2.at[NPAD:NPAD + N_NODES].set(x[:, HALF:])

    pad = jnp.full((EPAD - N_EDGES,), GARBAGE, jnp.int32)
    src_p = jnp.concatenate([g[0], pad])
    dst_p = jnp.concatenate([g[1], pad])
    src_p = jnp.stack([src_p, src_p + NPAD])

    out = _sc_diffuse(src_p, dst_p, x2, None)
    return jnp.concatenate([out[:N_NODES], out[NPAD:NPAD + N_NODES]], axis=1)
